# baseline (speedup 1.0000x reference)
import jax
import jax.numpy as jnp
from jax import lax
from jax.experimental import pallas as pl
from jax.experimental.pallas import tpu as pltpu
from jax.experimental.pallas import tpu_sc as plsc

N = 10000
D = 128
E = 80000
L1 = 64
L2 = 64

NPAD = 10240
EPAD = 98304
TRASH = N
NC, NS, LANES = 2, 16, 16
NTILES = NC * NS
EPT = EPAD // NTILES
CH = 128
NCHUNK = EPT // CH
RPT = NPAD // NS
NUV = 10112
ZW = 160

_mesh = plsc.VectorSubcoreMesh(core_axis_name="c", subcore_axis_name="s")


def _iota16():
    return lax.broadcasted_iota(jnp.int32, (LANES,), 0)


def _make_agg(with_cnt):
    out_type = [jax.ShapeDtypeStruct((NC, NPAD, D), jnp.float32)]
    scratch = [
        pltpu.VMEM_SHARED((NPAD, D), jnp.float32),
        pltpu.VMEM((NCHUNK, CH), jnp.int32),
        pltpu.VMEM((NCHUNK, CH), jnp.int32),
        pltpu.VMEM((CH, D), jnp.float32),
        pltpu.VMEM((CH, D), jnp.float32),
        pltpu.SemaphoreType.DMA,
        pltpu.SemaphoreType.DMA,
    ]
    if with_cnt:
        out_type.append(jax.ShapeDtypeStruct((NTILES, NUV), jnp.float32))
        scratch += [
            pltpu.VMEM((NUV,), jnp.float32),
        ]

    def body(*refs):
        if with_cnt:
            (table, rows_h, cols_h, zrow_h, zcnt_h,
             out_s, out_c, acc, rslab, cslab, ga, gb,
             sema, semb, hist) = refs
        else:
            (table, rows_h, cols_h, zrow_h,
             out_s, acc, rslab, cslab, ga, gb, sema, semb) = refs
        cid = lax.axis_index("c")
        sid = lax.axis_index("s")
        wid = cid * NS + sid
        row0 = wid * NCHUNK

        pltpu.sync_copy(zrow_h, acc.at[pl.ds(sid * RPT, RPT)])
        if with_cnt:
            pltpu.sync_copy(zcnt_h, hist)
        pltpu.sync_copy(rows_h.at[pl.ds(row0, NCHUNK)], rslab)
        pltpu.sync_copy(cols_h.at[pl.ds(row0, NCHUNK)], cslab)
        plsc.subcore_barrier()
        ones16 = jnp.ones((LANES,), jnp.float32)

        pltpu.async_copy(table.at[cslab.at[0]], ga, sema)

        def step(k, buf, sem, nbuf, nsem):
            @pl.when(k + 1 < NCHUNK)
            def _():
                pltpu.async_copy(table.at[cslab.at[k + 1]], nbuf, nsem)
            for g in range(CH // LANES):
                sl = pl.ds(g * LANES, LANES)
                r = rslab[k, sl]
                c = cslab[k, sl]
                d = jnp.where(r == c, TRASH, r)
                rslab[k, sl] = d
                if with_cnt:
                    plsc.addupdate_scatter(hist, [d], ones16)
            pltpu.make_async_copy(table.at[cslab.at[k]], buf, sem).wait()
            pltpu.sync_copy(buf, acc.at[rslab.at[k]], add=True)

        def pair(m, carry):
            step(2 * m, ga, sema, gb, semb)
            step(2 * m + 1, gb, semb, ga, sema)
            return carry

        lax.fori_loop(0, NCHUNK // 2, pair, 0)
        plsc.subcore_barrier()

        sl = pl.ds(sid * RPT, RPT)
        pltpu.sync_copy(acc.at[sl], out_s.at[cid].at[sl])
        if with_cnt:
            pltpu.sync_copy(hist, out_c.at[wid])

    return pl.kernel(body, out_type=tuple(out_type), mesh=_mesh,
                     compiler_params=pltpu.CompilerParams(
                         needs_layout_passes=False,
                         disable_bounds_checks=True),
                     scratch_types=scratch)


_agg_cnt = _make_agg(True)
_agg_sum = _make_agg(False)


RB = 256
GRID = NPAD // RB


def _l2t(u):
    n = jnp.sqrt(jnp.sum(u * u, axis=1, keepdims=True))
    return jnp.tanh(u / jnp.maximum(n, 1e-12))


def _base_tc_body(sp_ref, cp_ref, sn_ref, cn_ref, x_ref,
                  wp_ref, bp_ref, wn_ref, bn_ref, out_ref):
    x = x_ref[...]
    sp = sp_ref[0] + sp_ref[1]
    cp = jnp.maximum(cp_ref[...], 1.0)
    aggp = sp / cp
    up = (jnp.dot(aggp, wp_ref[0:D, :], preferred_element_type=jnp.float32)
          + jnp.dot(x, wp_ref[D:2 * D, :], preferred_element_type=jnp.float32)
          + bp_ref[...])
    sn = sn_ref[0] + sn_ref[1]
    cn = jnp.maximum(cn_ref[...], 1.0)
    aggn = sn / cn
    un = (jnp.dot(aggn, wn_ref[0:D, :], preferred_element_type=jnp.float32)
          + jnp.dot(x, wn_ref[D:2 * D, :], preferred_element_type=jnp.float32)
          + bn_ref[...])
    out_ref[...] = jnp.concatenate([_l2t(up), _l2t(un)], axis=1)


def _deep_tc_body(spd_ref, snd_ref, cp_ref, cn_ref, h_ref,
                  wp_ref, bp_ref, wn_ref, bn_ref, rw_ref, out_ref, uv_ref):
    h = h_ref[...]
    hp = h[:, :L1]
    hn = h[:, L1:]
    sp = spd_ref[0] + spd_ref[1]
    sn = snd_ref[0] + snd_ref[1]
    cp1 = cp_ref[...] + 1.0
    cn1 = cn_ref[...] + 1.0

    def head(o1, o2, xs, w_ref, b_ref):
        u = (jnp.dot(o1, w_ref[0:L1, :], preferred_element_type=jnp.float32)
             + jnp.dot(o2, w_ref[L1:2 * L1, :], preferred_element_type=jnp.float32)
             + jnp.dot(xs, w_ref[2 * L1:3 * L1, :], preferred_element_type=jnp.float32)
             + b_ref[...])
        return _l2t(u)

    zp = head((sp[:, :L1] + hp) / cp1, (sn[:, L1:] + hn) / cn1, hp, wp_ref, bp_ref)
    zn = head((sp[:, L1:] + hn) / cp1, (sn[:, :L1] + hp) / cn1, hn, wn_ref, bn_ref)
    z = jnp.concatenate([zp, zn], axis=1)
    out_ref[...] = z
    uv_ref[...] = jnp.dot(z, rw_ref[...], preferred_element_type=jnp.float32)


def _row_spec(shape3):
    return pl.BlockSpec((NC, RB) + shape3, lambda i: (0, i, 0))


def _col_spec():
    return pl.BlockSpec((RB, 1), lambda i: (i, 0))


def _full_spec(shape):
    nd = len(shape)
    return pl.BlockSpec(shape, lambda i, _n=nd: (0,) * _n)


_base_tc = pl.pallas_call(
    _base_tc_body,
    grid=(GRID,),
    in_specs=[
        _row_spec((D,)), _col_spec(), _row_spec((D,)), _col_spec(),
        pl.BlockSpec((RB, D), lambda i: (i, 0)),
        _full_spec((2 * D, L1)), _full_spec((1, L1)),
        _full_spec((2 * D, L1)), _full_spec((1, L1)),
    ],
    out_specs=pl.BlockSpec((RB, 2 * L1), lambda i: (i, 0)),
    out_shape=jax.ShapeDtypeStruct((NPAD, 2 * L1), jnp.float32),
)

_deep_tc = pl.pallas_call(
    _deep_tc_body,
    grid=(GRID,),
    in_specs=[
        _row_spec((D,)), _row_spec((D,)), _col_spec(), _col_spec(),
        pl.BlockSpec((RB, 2 * L1), lambda i: (i, 0)),
        _full_spec((3 * L1, L2)), _full_spec((1, L2)),
        _full_spec((3 * L1, L2)), _full_spec((1, L2)),
        _full_spec((D, 8)),
    ],
    out_specs=(pl.BlockSpec((RB, D), lambda i: (i, 0)),
               pl.BlockSpec((RB, 8), lambda i: (i, 0))),
    out_shape=(jax.ShapeDtypeStruct((NPAD, D), jnp.float32),
               jax.ShapeDtypeStruct((NPAD, 8), jnp.float32)),
)


_LN2 = 0.6931471805599453


def _fastlog(s):
    bits = lax.bitcast_convert_type(s, jnp.int32)
    e = ((bits >> 23) & 0xFF).astype(jnp.float32) - 127.0
    m = lax.bitcast_convert_type((bits & 0x7FFFFF) | 0x3F800000, jnp.float32)
    t = (m - 1.0) / (m + 1.0)
    t2 = t * t
    poly = 1.0 + t2 * (1.0 / 3.0 + t2 * (1.0 / 5.0 + t2 * (1.0 / 7.0 + t2 / 9.0)))
    return e * _LN2 + 2.0 * t * poly


def _loss_body(z_h, uvp0_h, uvp1_h, uvp2_h, pi_h, pj_h, pk_h, ni_h, nj_h, nk_h, t6_h,
               out_reg, out_relu,
               islab, jslab, kslab, bi, bj, bk, uv0, uv1, uv2,
               t0s, t1s, t2s, accb, sem):
    cid = lax.axis_index("c")
    sid = lax.axis_index("s")
    wid = cid * NS + sid
    row0 = wid * NCHUNK
    iota = _iota16()
    z16 = jnp.zeros((LANES,), jnp.float32)
    uvt = (uv0, uv1, uv2)
    tbs = (t0s, t1s, t2s)
    for c, uh in enumerate((uvp0_h, uvp1_h, uvp2_h)):
        pltpu.sync_copy(uh.at[pl.ds(0, NUV)], uvt[c])

    acc_reg = z16
    srelu = jnp.float32(0.0)
    fams = [
        (pi_h, pj_h, pk_h, (0, 4, 5), 1.0),
        (ni_h, nj_h, nk_h, (1, 2, 3), -1.0),
    ]
    for ih, jh, kh, blocks, sign in fams:
        pltpu.sync_copy(ih.at[pl.ds(row0, NCHUNK)], islab)
        pltpu.sync_copy(jh.at[pl.ds(row0, NCHUNK)], jslab)
        pltpu.sync_copy(kh.at[pl.ds(row0, NCHUNK)], kslab)
        for o, blk in enumerate(blocks):
            pltpu.sync_copy(t6_h.at[blk].at[pl.ds(row0, NCHUNK)], tbs[o])

        def chunk(k, carry, sign=sign):
            acc_reg, srelu = carry
            d1 = pltpu.async_copy(z_h.at[islab.at[k]], bi, sem)
            d2 = pltpu.async_copy(z_h.at[jslab.at[k]], bj, sem)
            d3 = pltpu.async_copy(z_h.at[kslab.at[k]], bk, sem)
            d1.wait()
            d2.wait()
            d3.wait()
            base = wid * EPT + k * CH

            def group(g, carry2):
                acc_reg, srelu = carry2
                sl = pl.ds(g * LANES, LANES)
                na = islab[k, sl]
                nb = jslab[k, sl]
                nc_ = kslab[k, sl]
                validf = jnp.where(base + g * LANES + iota < E, 1.0, 0.0)
                for o, (xa, xb) in enumerate(((na, nb), (na, nc_), (nb, nc_))):
                    p = []
                    for ci in range(3):
                        wa = plsc.load_gather(uvt[ci], [xa])
                        wb = plsc.load_gather(uvt[ci], [xb])
                        uu = lax.bitcast_convert_type(wa << 16, jnp.float32)
                        vv = lax.bitcast_convert_type(
                            wb & jnp.int32(-65536), jnp.float32)
                        p.append(uu + vv)
                    m = jnp.maximum(p[0], jnp.maximum(p[1], p[2]))
                    ssum = (jnp.exp(p[0] - m) + jnp.exp(p[1] - m)
                            + jnp.exp(p[2] - m))
                    ls = m + _fastlog(ssum)
                    t = tbs[o][k, sl]
                    pt = jnp.where(t == 0, p[0], jnp.where(t == 1, p[1], p[2]))
                    acc_reg = acc_reg + (ls - pt) * validf
                for e in range(LANES):
                    row = g * LANES + e
                    vij = z16
                    vik = z16
                    for q in range(D // LANES):
                        qsl = pl.ds(q * LANES, LANES)
                        a = bi[row, qsl]
                        b = bj[row, qsl]
                        c = bk[row, qsl]
                        db_ = a - b
                        dc_ = a - c
                        vij = vij + db_ * db_
                        vik = vik + dc_ * dc_
                    dij = lax.reduce_sum_p.bind(vij, axes=(0,))
                    dik = lax.reduce_sum_p.bind(vik, axes=(0,))
                    srelu = srelu + jnp.maximum(sign * (dij - dik),
                                                jnp.float32(0.0))
                return acc_reg, srelu

            return lax.fori_loop(0, CH // LANES, group, (acc_reg, srelu))

        acc_reg, srelu = lax.fori_loop(0, NCHUNK, chunk, (acc_reg, srelu))

    accb[0, :] = acc_reg
    pltpu.sync_copy(accb, out_reg.at[pl.ds(wid, 1)])
    accb[0, :] = jnp.where(iota == 0, srelu, 0.0)
    pltpu.sync_copy(accb, out_relu.at[pl.ds(wid, 1)])


_loss_sc = pl.kernel(
    _loss_body,
    out_type=(jax.ShapeDtypeStruct((NTILES, 16), jnp.float32),
              jax.ShapeDtypeStruct((NTILES, 16), jnp.float32)),
    mesh=_mesh,
    compiler_params=pltpu.CompilerParams(needs_layout_passes=False,
                                         disable_bounds_checks=True),
    scratch_types=[
        pltpu.VMEM((NCHUNK, CH), jnp.int32),
        pltpu.VMEM((NCHUNK, CH), jnp.int32),
        pltpu.VMEM((NCHUNK, CH), jnp.int32),
        pltpu.VMEM((CH, D), jnp.float32),
        pltpu.VMEM((CH, D), jnp.float32),
        pltpu.VMEM((CH, D), jnp.float32),
        pltpu.VMEM((NUV,), jnp.int32),
        pltpu.VMEM((NUV,), jnp.int32),
        pltpu.VMEM((NUV,), jnp.int32),
        pltpu.VMEM((NCHUNK, CH), jnp.int32),
        pltpu.VMEM((NCHUNK, CH), jnp.int32),
        pltpu.VMEM((NCHUNK, CH), jnp.int32),
        pltpu.VMEM((1, 16), jnp.float32),
        pltpu.SemaphoreType.DMA,
    ],
)


def kernel(X, positive_edges, negative_edges, target, pos_surrogates,
           neg_surrogates, W_pos_base, b_pos_base, W_neg_base, b_neg_base,
           W_pos_deep, b_pos_deep, W_neg_deep, b_neg_deep,
           regression_weights):
    f32 = jnp.float32
    padE = lambda a: jnp.pad(a, (0, EPAD - E)).reshape(EPAD // CH, CH)
    rp, cp_ = padE(positive_edges[0]), padE(positive_edges[1])
    rn, cn_ = padE(negative_edges[0]), padE(negative_edges[1])
    pk = padE(pos_surrogates)
    nk = padE(neg_surrogates)
    t6 = jnp.pad(target.reshape(6, E),
                 ((0, 0), (0, EPAD - E))).reshape(6, EPAD // CH, CH)
    Xp = jnp.pad(X, ((0, NPAD - N), (0, 0)))

    zrow = jnp.zeros((RPT, D), f32)
    zcnt = jnp.zeros((NUV,), f32)

    sp, cp32 = _agg_cnt(Xp, rp, cp_, zrow, zcnt)
    sn, cn32 = _agg_cnt(Xp, rn, cn_, zrow, zcnt)
    cpc = jnp.pad(jnp.sum(cp32, axis=0), (0, NPAD - NUV))[:, None]
    cnc = jnp.pad(jnp.sum(cn32, axis=0), (0, NPAD - NUV))[:, None]
    H = _base_tc(sp, cpc, sn, cnc, Xp,
                 W_pos_base, b_pos_base.reshape(1, L1),
                 W_neg_base, b_neg_base.reshape(1, L1))
    (spd,) = _agg_sum(H, rp, cp_, zrow)
    (snd,) = _agg_sum(H, rn, cn_, zrow)
    rw8 = jnp.pad(jnp.concatenate(
        [regression_weights[:D], regression_weights[D:]], axis=1),
        ((0, 0), (0, 2)))
    Z, UVo = _deep_tc(spd, snd, cpc, cnc, H,
                      W_pos_deep, b_pos_deep.reshape(1, L2),
                      W_neg_deep, b_neg_deep.reshape(1, L2), rw8)
    ub = lax.bitcast_convert_type(
        UVo[:, :3].astype(jnp.bfloat16), jnp.uint16).astype(jnp.int32)
    vb = lax.bitcast_convert_type(
        UVo[:, 3:6].astype(jnp.bfloat16), jnp.uint16).astype(jnp.int32)
    uvp = (vb << 16) | ub
    reg, relu = _loss_sc(Z, uvp[:, 0], uvp[:, 1], uvp[:, 2],
                         rp, cp_, pk, rn, cn_, nk, t6)
    z = Z[:N]
    loss = jnp.sum(reg) / (6.0 * E) + jnp.sum(relu) / E
    return loss, z
